# baseline (speedup 1.0000x reference)
import functools

import jax
import jax.numpy as jnp
from jax import lax
from jax.experimental import pallas as pl
from jax.experimental.pallas import tpu as pltpu
from jax.experimental.pallas import tpu_sc as plsc

N_NODES = 10000
D = 128
N_EDGES = 320000
CHUNK = 128
N_CHUNKS = N_EDGES // CHUNK
N_SUBCORES = 16
NCH = 160
N_CHUNKS_PAD = N_SUBCORES * NCH
N_NODES_PAD = N_NODES + 8
KB = 32
NBATCH = NCH // KB
SLAB_ROWS = 1000
N_SLABS = N_NODES // SLAB_ROWS

ROW_BLK = 2000


def _tc_body(uf, itf, wu, bu, wi, bi, wum, bum, wim, bim,
             uemb, gu, iemb, gi):
    dn = (((1,), (1,)), ((), ()))
    u = uf[...]
    t = itf[...]
    uemb[...] = lax.dot_general(u, wu[...], dn, preferred_element_type=jnp.float32) + bu[...]
    gu[...] = lax.dot_general(u, wum[...], dn, preferred_element_type=jnp.float32) + bum[...]
    iemb[...] = lax.dot_general(t, wi[...], dn, preferred_element_type=jnp.float32) + bi[...]
    gi[...] = lax.dot_general(t, wim[...], dn, preferred_element_type=jnp.float32) + bim[...]


def _tc_stage(uf, itf, Wu, bu, Wi, bi, Wum, bum, Wim, bim):
    blk = pl.BlockSpec((ROW_BLK, D), lambda i: (i, 0))
    full = pl.BlockSpec((D, D), lambda i: (0, 0))
    bias = pl.BlockSpec((1, D), lambda i: (0, 0))
    out_sds = jax.ShapeDtypeStruct((N_NODES, D), jnp.float32)
    return pl.pallas_call(
        _tc_body,
        grid=(N_NODES // ROW_BLK,),
        in_specs=[blk, blk, full, bias, full, bias, full, bias, full, bias],
        out_specs=[blk, blk, blk, blk],
        out_shape=[out_sds, out_sds, out_sds, out_sds],
    )(uf, itf, Wu, bu.reshape(1, D), Wi, bi.reshape(1, D),
      Wum, bum.reshape(1, D), Wim, bim.reshape(1, D))


def _run_chunks(start_chunk, src_g, src_idx_hbm, dst_idx_hbm,
                accum, idxs_s, idxs_d, rows, gsem, ssem, isem):
    def fire_gather(b, buf, t):
        pltpu.async_copy(src_g.at[idxs_s[buf].at[t]], rows[b], gsem[b])

    def wait_gather(b, buf, t):
        pltpu.make_async_copy(src_g.at[idxs_s[buf].at[t]], rows[b], gsem[b]).wait()

    def fire_scatter(b, buf, t):
        pltpu.async_copy(rows[b], accum.at[idxs_d[buf].at[t]], ssem[b], add=True)

    def wait_scatter(b, buf, t):
        pltpu.make_async_copy(rows[b], accum.at[idxs_d[buf].at[t]], ssem[b]).wait()

    def idx_batch_descs(p, buf):
        base = pl.ds(start_chunk + p * KB, KB)
        return (pltpu.make_async_copy(src_idx_hbm.at[base], idxs_s[buf], isem),
                pltpu.make_async_copy(dst_idx_hbm.at[base], idxs_d[buf], isem))

    pltpu.sync_copy(src_idx_hbm.at[pl.ds(start_chunk, KB)], idxs_s[0])
    pltpu.sync_copy(dst_idx_hbm.at[pl.ds(start_chunk, KB)], idxs_d[0])
    fire_gather(0, 0, 0)

    for p in range(NBATCH):
        buf, nbuf = p % 2, 1 - p % 2

        if p > 0:
            wait_scatter(1, nbuf, KB - 1)
        if p + 1 < NBATCH:
            for desc in idx_batch_descs(p + 1, nbuf):
                desc.start()
        fire_gather(1, buf, 1)
        wait_gather(0, buf, 0)
        fire_scatter(0, buf, 0)

        def pair(m, carry, buf=buf):
            for tb in (0, 1):
                t = 1 + 2 * m + tb
                b, nb = (1 + tb) % 2, tb % 2
                wait_scatter(nb, buf, t - 1)
                fire_gather(nb, buf, t + 1)
                wait_gather(b, buf, t)
                fire_scatter(b, buf, t)
            return carry

        lax.fori_loop(0, (KB - 2) // 2, pair, 0)

        wait_scatter(0, buf, KB - 2)
        if p + 1 < NBATCH:
            for desc in idx_batch_descs(p + 1, nbuf):
                desc.wait()
            fire_gather(0, nbuf, 0)
        wait_gather(1, buf, KB - 1)
        fire_scatter(1, buf, KB - 1)

    wait_scatter(1, (NBATCH - 1) % 2, KB - 1)


def _sc_direction(src_g, init_emb, src_idx_hbm, dst_idx_hbm, out_hbm,
                  accum, idxs_s, idxs_d, rows, gsem, ssem, isem, s):
    slab = pl.ds(s * SLAB_ROWS, SLAB_ROWS)

    @pl.when(s < N_SLABS)
    def _init():
        pltpu.sync_copy(init_emb.at[slab], accum.at[slab])

    plsc.subcore_barrier()

    _run_chunks(s * NCH, src_g, src_idx_hbm, dst_idx_hbm,
                accum, idxs_s, idxs_d, rows, gsem, ssem, isem)

    plsc.subcore_barrier()

    @pl.when(s < N_SLABS)
    def _writeout():
        pltpu.sync_copy(accum.at[slab], out_hbm.at[slab])


def _sc_body(gu, gi, uemb, iemb, uidx, iidx, user_out, item_out,
             accum, idxs_s0, idxs_s1, idxs_d0, idxs_d1, rows0, rows1,
             gsem0, gsem1, ssem0, ssem1, isem):
    c = lax.axis_index("c")
    s = lax.axis_index("s")
    idxs_s, idxs_d = (idxs_s0, idxs_s1), (idxs_d0, idxs_d1)
    rows, gsem, ssem = (rows0, rows1), (gsem0, gsem1), (ssem0, ssem1)

    @pl.when(c == 0)
    def _items():
        _sc_direction(gu, iemb, uidx, iidx, item_out,
                      accum, idxs_s, idxs_d, rows, gsem, ssem, isem, s)

    @pl.when(c == 1)
    def _users():
        _sc_direction(gi, uemb, iidx, uidx, user_out,
                      accum, idxs_s, idxs_d, rows, gsem, ssem, isem, s)


@functools.cache
def _sc_stage():
    return pl.kernel(
        _sc_body,
        out_type=[jax.ShapeDtypeStruct((N_NODES, D), jnp.float32),
                  jax.ShapeDtypeStruct((N_NODES, D), jnp.float32)],
        mesh=plsc.VectorSubcoreMesh(core_axis_name="c", subcore_axis_name="s"),
        scratch_types=[
            pltpu.VMEM_SHARED((N_NODES_PAD, D), jnp.float32),
            pltpu.VMEM((KB, CHUNK), jnp.int32),
            pltpu.VMEM((KB, CHUNK), jnp.int32),
            pltpu.VMEM((KB, CHUNK), jnp.int32),
            pltpu.VMEM((KB, CHUNK), jnp.int32),
            pltpu.VMEM((CHUNK, D), jnp.float32),
            pltpu.VMEM((CHUNK, D), jnp.float32),
            pltpu.SemaphoreType.DMA,
            pltpu.SemaphoreType.DMA,
            pltpu.SemaphoreType.DMA,
            pltpu.SemaphoreType.DMA,
            pltpu.SemaphoreType.DMA,
        ],
    )


def kernel(user_features, item_features, edge_index, Wu, bu, Wi, bi,
           Wum, bum, Wim, bim):
    uemb, gu, iemb, gi = _tc_stage(user_features, item_features,
                                   Wu, bu, Wi, bi, Wum, bum, Wim, bim)
    gu = jnp.pad(gu, ((0, N_NODES_PAD - N_NODES), (0, 0)))
    gi = jnp.pad(gi, ((0, N_NODES_PAD - N_NODES), (0, 0)))
    pad_e = N_CHUNKS_PAD * CHUNK - N_EDGES
    u_idx = jnp.pad(edge_index[0].astype(jnp.int32), (0, pad_e),
                    constant_values=N_NODES).reshape(N_CHUNKS_PAD, CHUNK)
    i_idx = jnp.pad(edge_index[1].astype(jnp.int32), (0, pad_e),
                    constant_values=N_NODES).reshape(N_CHUNKS_PAD, CHUNK)
    user_out, item_out = _sc_stage()(gu, gi, uemb, iemb, u_idx, i_idx)
    return (user_out, item_out)


# speedup vs baseline: 2.5140x; 2.5140x over previous
import functools

import jax
import jax.numpy as jnp
from jax import lax
from jax.experimental import pallas as pl
from jax.experimental.pallas import tpu as pltpu
from jax.experimental.pallas import tpu_sc as plsc

N_NODES = 10000
D = 128
N_EDGES = 320000
CHUNK = 128
N_CHUNKS = N_EDGES // CHUNK
N_SUBCORES = 16
FULL_ROUNDS = N_CHUNKS // N_SUBCORES
TAIL = N_CHUNKS - FULL_ROUNDS * N_SUBCORES
SLAB_ROWS = 1000
N_SLABS = N_NODES // SLAB_ROWS

ROW_BLK = 2000


def _tc_body(uf, itf, wu, bu, wi, bi, wum, bum, wim, bim,
             uemb, gu, iemb, gi):
    dn = (((1,), (1,)), ((), ()))
    u = uf[...]
    t = itf[...]
    uemb[...] = lax.dot_general(u, wu[...], dn, preferred_element_type=jnp.float32) + bu[...]
    gu[...] = lax.dot_general(u, wum[...], dn, preferred_element_type=jnp.float32) + bum[...]
    iemb[...] = lax.dot_general(t, wi[...], dn, preferred_element_type=jnp.float32) + bi[...]
    gi[...] = lax.dot_general(t, wim[...], dn, preferred_element_type=jnp.float32) + bim[...]


def _tc_stage(uf, itf, Wu, bu, Wi, bi, Wum, bum, Wim, bim):
    blk = pl.BlockSpec((ROW_BLK, D), lambda i: (i, 0))
    full = pl.BlockSpec((D, D), lambda i: (0, 0))
    bias = pl.BlockSpec((1, D), lambda i: (0, 0))
    out_sds = jax.ShapeDtypeStruct((N_NODES, D), jnp.float32)
    return pl.pallas_call(
        _tc_body,
        grid=(N_NODES // ROW_BLK,),
        in_specs=[blk, blk, full, bias, full, bias, full, bias, full, bias],
        out_specs=[blk, blk, blk, blk],
        out_shape=[out_sds, out_sds, out_sds, out_sds],
    )(uf, itf, Wu, bu.reshape(1, D), Wi, bi.reshape(1, D),
      Wum, bum.reshape(1, D), Wim, bim.reshape(1, D))


def _run_chunks(s, src_g, src_idx_hbm, dst_idx_hbm,
                accum, idx_s, idx_d, rows, gsem, ssem, isem):
    n = FULL_ROUNDS

    def base(j):
        return (s + j * N_SUBCORES) * CHUNK

    def fire_idx(q, j):
        pltpu.async_copy(src_idx_hbm.at[pl.ds(base(j), CHUNK)], idx_s[q], isem[q])
        pltpu.async_copy(dst_idx_hbm.at[pl.ds(base(j), CHUNK)], idx_d[q], isem[q])

    def wait_idx(q, j):
        pltpu.make_async_copy(src_idx_hbm.at[pl.ds(base(j), CHUNK)], idx_s[q], isem[q]).wait()
        pltpu.make_async_copy(dst_idx_hbm.at[pl.ds(base(j), CHUNK)], idx_d[q], isem[q]).wait()

    def fire_gather(r, q):
        pltpu.async_copy(src_g.at[idx_s[q]], rows[r], gsem[r])

    def wait_gather(r, q):
        pltpu.make_async_copy(src_g.at[idx_s[q]], rows[r], gsem[r]).wait()

    def fire_scatter(r, q):
        pltpu.async_copy(rows[r], accum.at[idx_d[q]], ssem[r], add=True)

    def wait_scatter(r, q):
        pltpu.make_async_copy(rows[r], accum.at[idx_d[q]], ssem[r]).wait()

    def step(j, u):
        q, r = u, u % 2
        if not (isinstance(j, int) and j == 0):
            wait_scatter((r + 1) % 2, (q + 3) % 4)
        if not isinstance(j, int) or j + 2 < n:
            fire_idx((q + 2) % 4, j + 2)
        if not isinstance(j, int) or j + 1 < n:
            wait_idx((q + 1) % 4, j + 1)
            fire_gather((r + 1) % 2, (q + 1) % 4)
        wait_gather(r, q)
        fire_scatter(r, q)

    fire_idx(0, 0)
    fire_idx(1, 1)
    wait_idx(0, 0)
    fire_gather(0, 0)

    for j in range(4):
        step(j, j % 4)

    def quad(m, carry):
        for u in range(4):
            step(4 + 4 * m + u, u)
        return carry

    n_quads = (n - 8) // 4
    lax.fori_loop(0, n_quads, quad, 0)

    for j in range(n - 4, n):
        step(j, j % 4)

    wait_scatter((n - 1) % 2, (n - 1) % 4)


def _run_tail_chunk(s, src_g, src_idx_hbm, dst_idx_hbm,
                    accum, idx_s, idx_d, rows, gsem, ssem, isem):
    b = (s + FULL_ROUNDS * N_SUBCORES) * CHUNK
    pltpu.sync_copy(src_idx_hbm.at[pl.ds(b, CHUNK)], idx_s[0])
    pltpu.sync_copy(dst_idx_hbm.at[pl.ds(b, CHUNK)], idx_d[0])
    pltpu.async_copy(src_g.at[idx_s[0]], rows[0], gsem[0]).wait()
    pltpu.sync_copy(rows[0], accum.at[idx_d[0]], add=True)


def _sc_direction(src_g, init_emb, src_idx_hbm, dst_idx_hbm, out_hbm,
                  accum, idxs_s, idxs_d, rows, gsem, ssem, isem, s):
    slab = pl.ds(s * SLAB_ROWS, SLAB_ROWS)

    @pl.when(s < N_SLABS)
    def _init():
        pltpu.sync_copy(init_emb.at[slab], accum.at[slab])

    plsc.subcore_barrier()

    _run_chunks(s, src_g, src_idx_hbm, dst_idx_hbm,
                accum, idxs_s, idxs_d, rows, gsem, ssem, isem)

    @pl.when(s < TAIL)
    def _tail():
        _run_tail_chunk(s, src_g, src_idx_hbm, dst_idx_hbm,
                        accum, idxs_s, idxs_d, rows, gsem, ssem, isem)

    plsc.subcore_barrier()

    @pl.when(s < N_SLABS)
    def _writeout():
        pltpu.sync_copy(accum.at[slab], out_hbm.at[slab])


def _sc_body(gu, gi, uemb, iemb, uidx, iidx, user_out, item_out,
             accum, is0, is1, is2, is3, id0, id1, id2, id3, rows0, rows1,
             gsem0, gsem1, ssem0, ssem1, isem0, isem1, isem2, isem3):
    c = lax.axis_index("c")
    s = lax.axis_index("s")
    idxs_s, idxs_d = (is0, is1, is2, is3), (id0, id1, id2, id3)
    rows, gsem, ssem = (rows0, rows1), (gsem0, gsem1), (ssem0, ssem1)
    isem = (isem0, isem1, isem2, isem3)

    @pl.when(c == 0)
    def _items():
        _sc_direction(gu, iemb, uidx, iidx, item_out,
                      accum, idxs_s, idxs_d, rows, gsem, ssem, isem, s)

    @pl.when(c == 1)
    def _users():
        _sc_direction(gi, uemb, iidx, uidx, user_out,
                      accum, idxs_s, idxs_d, rows, gsem, ssem, isem, s)


@functools.cache
def _sc_stage():
    return pl.kernel(
        _sc_body,
        out_type=[jax.ShapeDtypeStruct((N_NODES, D), jnp.float32),
                  jax.ShapeDtypeStruct((N_NODES, D), jnp.float32)],
        mesh=plsc.VectorSubcoreMesh(core_axis_name="c", subcore_axis_name="s"),
        scratch_types=(
            [pltpu.VMEM_SHARED((N_NODES, D), jnp.float32)]
            + [pltpu.VMEM((CHUNK,), jnp.int32)] * 8
            + [pltpu.VMEM((CHUNK, D), jnp.float32)] * 2
            + [pltpu.SemaphoreType.DMA] * 8
        ),
    )


def kernel(user_features, item_features, edge_index, Wu, bu, Wi, bi,
           Wum, bum, Wim, bim):
    uemb, gu, iemb, gi = _tc_stage(user_features, item_features,
                                   Wu, bu, Wi, bi, Wum, bum, Wim, bim)
    u_idx = edge_index[0].astype(jnp.int32)
    i_idx = edge_index[1].astype(jnp.int32)
    user_out, item_out = _sc_stage()(gu, gi, uemb, iemb, u_idx, i_idx)
    return (user_out, item_out)
